# baseline (speedup 1.0000x reference)
import jax
import jax.numpy as jnp
from jax import lax
from jax.experimental import pallas as pl
from jax.experimental.pallas import tpu as pltpu
from jax.experimental.pallas import tpu_sc as plsc

N = 50000
E = 800000
DIM = 32

BN = 400
GRID = N // BN

EPB = 128
GRP = 16
EPG = GRP * EPB
NG = -(-E // (EPG * 16)) * 16
EPAD = NG * EPG
NTILE = 16
HN = N // 2
RPT = 1568
HNPAD = RPT * NTILE

_MESH = dict(
    mesh=plsc.VectorSubcoreMesh(
        core_axis_name="c", subcore_axis_name="s", num_cores=2,
        num_subcores=NTILE),
    compiler_params=pltpu.CompilerParams(use_tc_tiling_on_sc=False))


def _zero_spmem(zbuf, spm, row0):
    def zrow(i, _):
        zbuf[i, pl.ds(0, 16)] = jnp.zeros((16,), jnp.float32)
        zbuf[i, pl.ds(16, 16)] = jnp.zeros((16,), jnp.float32)
        return 0
    lax.fori_loop(0, 128, zrow, 0)
    nfull = RPT // 128
    rem = RPT - nfull * 128
    def zcp(k, _):
        pltpu.sync_copy(zbuf, spm.at[pl.ds(row0 + k * 128, 128)])
        return 0
    lax.fori_loop(0, nfull, zcp, 0)
    pltpu.sync_copy(zbuf.at[pl.ds(0, rem)], spm.at[pl.ds(row0 + nfull * 128, rem)])


def _sc_agg(src3d, dst4d, x):
    ngrp = NG // NTILE

    def body(src_hbm, dst_hbm, x_hbm, out_hbm,
             srcv, dstv, rows, zbuf, spm, *sems):
        c = lax.axis_index("c")
        s = lax.axis_index("s")
        row0 = s * RPT
        _zero_spmem(zbuf, spm, row0)
        plsc.subcore_barrier()
        grp_base = s * ngrp

        def group(g, _):
            idx = grp_base + g
            pltpu.sync_copy(src_hbm.at[idx], srcv)
            pltpu.sync_copy(dst_hbm.at[c].at[idx], dstv)
            descs = [pltpu.async_copy(x_hbm.at[srcv.at[j]], rows.at[j],
                                      sems[j]) for j in range(GRP)]
            for j in range(GRP):
                descs[j].wait()
                pltpu.sync_copy(rows.at[j], spm.at[dstv.at[j]], add=True)
            return 0
        lax.fori_loop(0, ngrp, group, 0)

        plsc.subcore_barrier()
        pltpu.sync_copy(spm.at[pl.ds(row0, RPT)],
                        out_hbm.at[c].at[pl.ds(row0, RPT)])

    f = pl.kernel(
        body,
        out_type=jax.ShapeDtypeStruct((2, HNPAD, DIM), jnp.float32),
        scratch_types=[
            pltpu.VMEM((GRP, EPB), jnp.int32),
            pltpu.VMEM((GRP, EPB), jnp.int32),
            pltpu.VMEM((GRP, EPB, DIM), jnp.float32),
            pltpu.VMEM((128, DIM), jnp.float32),
            pltpu.VMEM_SHARED((HNPAD, DIM), jnp.float32),
        ] + [pltpu.SemaphoreType.DMA] * GRP,
        **_MESH,
    )
    return f(src3d, dst4d, x)


def _row_spec():
    return pl.BlockSpec((BN, DIM), lambda i: (i, 0))


def _const_spec(shape):
    return pl.BlockSpec(shape, lambda i: tuple(0 for _ in shape))


def _embed_body(deg_ref, lab_ref, tdeg_ref, tlab_ref, wa_ref, wb_ref,
                lo_ref, hi_ref, t1_ref):
    d = deg_ref[...]
    l = lab_ref[...]
    oh_d = (d == lax.broadcasted_iota(jnp.int32, (BN, 64), 1)).astype(jnp.float32)
    oh_l = (l == lax.broadcasted_iota(jnp.int32, (BN, 16), 1)).astype(jnp.float32)
    lo = jnp.dot(oh_d, tdeg_ref[...], preferred_element_type=jnp.float32)
    hi = jnp.dot(oh_l, tlab_ref[...], preferred_element_type=jnp.float32)
    lo_ref[...] = lo
    hi_ref[...] = hi
    t1_ref[...] = (jnp.dot(lo, wa_ref[...], preferred_element_type=jnp.float32)
                   + jnp.dot(hi, wb_ref[...], preferred_element_type=jnp.float32))


def _embed(nd, nl, embed_deg, embed_lab, wa, wb):
    return pl.pallas_call(
        _embed_body,
        grid=(GRID,),
        in_specs=[pl.BlockSpec((BN, 1), lambda i: (i, 0)),
                  pl.BlockSpec((BN, 1), lambda i: (i, 0)),
                  _const_spec((64, DIM)),
                  _const_spec((16, DIM)),
                  _const_spec((DIM, DIM)),
                  _const_spec((DIM, DIM))],
        out_specs=[_row_spec(), _row_spec(), _row_spec()],
        out_shape=[jax.ShapeDtypeStruct((N, DIM), jnp.float32),
                   jax.ShapeDtypeStruct((N, DIM), jnp.float32),
                   jax.ShapeDtypeStruct((N, DIM), jnp.float32)],
    )(nd, nl, embed_deg, embed_lab, wa, wb)


def _accum_stats(st_ref, y):
    part = jnp.concatenate(
        [jnp.sum(y, 0, keepdims=True), jnp.sum(y * y, 0, keepdims=True)], 0)
    i = pl.program_id(0)

    @pl.when(i == 0)
    def _():
        st_ref[...] = part

    @pl.when(i > 0)
    def _():
        st_ref[...] += part


def _add_stats_body(t_ref, a_ref, b_ref, y_ref, st_ref):
    y = t_ref[...] + a_ref[...] + b_ref[...]
    y_ref[...] = y
    _accum_stats(st_ref, y)


def _add_stats(t, a, b):
    return pl.pallas_call(
        _add_stats_body,
        grid=(GRID,),
        in_specs=[_row_spec(), _row_spec(), _const_spec((1, DIM))],
        out_specs=[_row_spec(), _const_spec((2, DIM))],
        out_shape=[jax.ShapeDtypeStruct((N, DIM), jnp.float32),
                   jax.ShapeDtypeStruct((2, DIM), jnp.float32)],
    )(t, a, b)


def _bn_leaky(y, st_ref, g_ref, be_ref):
    stv = st_ref[...]
    mu = stv[0:1, :] * (1.0 / N)
    var = stv[1:2, :] * (1.0 / N) - mu * mu
    sc = g_ref[...] * lax.rsqrt(var + 1e-5)
    sh = be_ref[...] - mu * sc
    x = y * sc + sh
    return jnp.where(x >= 0, x, 0.01 * x)


def _bnact_mm_body(y_ref, st_ref, g_ref, be_ref, w_ref, x_ref, t_ref):
    x = _bn_leaky(y_ref[...], st_ref, g_ref, be_ref)
    x_ref[...] = x
    t_ref[...] = jnp.dot(x, w_ref[...], preferred_element_type=jnp.float32)


def _bnact_mm(y, st, g, be, w):
    return pl.pallas_call(
        _bnact_mm_body,
        grid=(GRID,),
        in_specs=[_row_spec(), _const_spec((2, DIM)),
                  _const_spec((1, DIM)), _const_spec((1, DIM)),
                  _const_spec((DIM, DIM))],
        out_specs=[_row_spec(), _row_spec()],
        out_shape=[jax.ShapeDtypeStruct((N, DIM), jnp.float32),
                   jax.ShapeDtypeStruct((N, DIM), jnp.float32)],
    )(y, st, g, be, w)


def _final_mm_body(y2_ref, st2_ref, g2_ref, be2_ref, tlo_ref, thi_ref, x1_ref,
                   fa_ref, fb_ref, fc_ref, fd_ref, b_ref, y3_ref, st3_ref):
    x2 = _bn_leaky(y2_ref[...], st2_ref, g2_ref, be2_ref)
    y3 = (jnp.dot(tlo_ref[...], fa_ref[...], preferred_element_type=jnp.float32)
          + jnp.dot(thi_ref[...], fb_ref[...], preferred_element_type=jnp.float32)
          + jnp.dot(x1_ref[...], fc_ref[...], preferred_element_type=jnp.float32)
          + jnp.dot(x2, fd_ref[...], preferred_element_type=jnp.float32)
          + b_ref[...])
    y3_ref[...] = y3
    _accum_stats(st3_ref, y3)


def _final_mm(y2, st2, g2, be2, tlo, thi, x1, fa, fb, fc, fd, b):
    return pl.pallas_call(
        _final_mm_body,
        grid=(GRID,),
        in_specs=[_row_spec(), _const_spec((2, DIM)),
                  _const_spec((1, DIM)), _const_spec((1, DIM)),
                  _row_spec(), _row_spec(), _row_spec(),
                  _const_spec((DIM, DIM)), _const_spec((DIM, DIM)),
                  _const_spec((DIM, DIM)), _const_spec((DIM, DIM)),
                  _const_spec((1, DIM))],
        out_specs=[_row_spec(), _const_spec((2, DIM))],
        out_shape=[jax.ShapeDtypeStruct((N, DIM), jnp.float32),
                   jax.ShapeDtypeStruct((2, DIM), jnp.float32)],
    )(y2, st2, g2, be2, tlo, thi, x1, fa, fb, fc, fd, b)


def _head_body(y3_ref, st3_ref, g_ref, be_ref, w_ref, b_ref, o_ref):
    x3 = _bn_leaky(y3_ref[...], st3_ref, g_ref, be_ref)
    o = jnp.dot(x3, w_ref[...], preferred_element_type=jnp.float32) + b_ref[...]
    o_ref[...] = 1.0 / (1.0 + jnp.exp(-o))


def _head(y3, st3, g, be, w, b):
    return pl.pallas_call(
        _head_body,
        grid=(GRID,),
        in_specs=[_row_spec(), _const_spec((2, DIM)),
                  _const_spec((1, DIM)), _const_spec((1, DIM)),
                  _const_spec((DIM, 1)), _const_spec((1, 1))],
        out_specs=pl.BlockSpec((BN, 1), lambda i: (i, 0)),
        out_shape=jax.ShapeDtypeStruct((N, 1), jnp.float32),
    )(y3, st3, g, be, w, b)


def kernel(node_deg, node_lab, edge_index, embed_deg, embed_lab, W1, b1, g1,
           be1, W2, b2, g2, be2, fcW1, fcb1, fcg, fcbe, fcW2, fcb2):
    nd = node_deg.astype(jnp.int32).reshape(N, 1)
    nl = node_lab.astype(jnp.int32).reshape(N, 1)

    pad = EPAD - E
    src = jnp.concatenate([edge_index[0].astype(jnp.int32),
                           jnp.zeros((pad,), jnp.int32)])
    dst = jnp.concatenate([edge_index[1].astype(jnp.int32),
                           jnp.full((pad,), N, jnp.int32)])
    dst0 = jnp.where(dst < HN, dst, HN)
    dst1 = jnp.where(dst >= HN, dst - HN, HN)
    src3d = src.reshape(NG, GRP, EPB)
    dst4d = jnp.stack([dst0, dst1]).reshape(2, NG, GRP, EPB)

    tlo, thi, t1 = _embed(nd, nl, embed_deg, embed_lab, W1[:DIM], W1[DIM:])

    agg1 = _sc_agg(src3d, dst4d, t1)
    a1 = jnp.concatenate([agg1[0, :HN], agg1[1, :HN]], 0)
    y1, st1 = _add_stats(t1, a1, b1.reshape(1, DIM))
    x1, t2 = _bnact_mm(y1, st1, g1.reshape(1, DIM), be1.reshape(1, DIM), W2)

    agg2 = _sc_agg(src3d, dst4d, t2)
    a2 = jnp.concatenate([agg2[0, :HN], agg2[1, :HN]], 0)
    y2, st2 = _add_stats(t2, a2, b2.reshape(1, DIM))

    y3, st3 = _final_mm(y2, st2, g2.reshape(1, DIM), be2.reshape(1, DIM),
                        tlo, thi, x1,
                        fcW1[0:DIM], fcW1[DIM:2 * DIM],
                        fcW1[2 * DIM:3 * DIM], fcW1[3 * DIM:],
                        fcb1.reshape(1, DIM))
    out = _head(y3, st3, fcg.reshape(1, DIM), fcbe.reshape(1, DIM),
                fcW2, fcb2.reshape(1, 1))
    return out.reshape(N)


# speedup vs baseline: 1.2982x; 1.2982x over previous
import jax
import jax.numpy as jnp
from jax import lax
from jax.experimental import pallas as pl
from jax.experimental.pallas import tpu as pltpu
from jax.experimental.pallas import tpu_sc as plsc

N = 50000
E = 800000
DIM = 32

BN = 400
GRID = N // BN

EPB = 128
GRP = 8
EPG = GRP * EPB
NG = -(-E // (EPG * 16)) * 16
EPAD = NG * EPG
NCH = 5
CH = NG // (16 * NCH)
NTILE = 16
HN = N // 2
RPT = 1568
HNPAD = RPT * NTILE

_MESH = dict(
    mesh=plsc.VectorSubcoreMesh(
        core_axis_name="c", subcore_axis_name="s", num_cores=2,
        num_subcores=NTILE),
    compiler_params=pltpu.CompilerParams(use_tc_tiling_on_sc=False))


def _zero_spmem(zbuf, spm, row0):
    def zrow(i, _):
        zbuf[i, pl.ds(0, 16)] = jnp.zeros((16,), jnp.float32)
        zbuf[i, pl.ds(16, 16)] = jnp.zeros((16,), jnp.float32)
        return 0
    lax.fori_loop(0, 128, zrow, 0)
    nfull = RPT // 128
    rem = RPT - nfull * 128
    def zcp(k, _):
        pltpu.sync_copy(zbuf, spm.at[pl.ds(row0 + k * 128, 128)])
        return 0
    lax.fori_loop(0, nfull, zcp, 0)
    pltpu.sync_copy(zbuf.at[pl.ds(0, rem)], spm.at[pl.ds(row0 + nfull * 128, rem)])


def _sc_agg(src3d, dst4d, x):
    ngrp = NG // NTILE

    def body(src_hbm, dst_hbm, x_hbm, out_hbm,
             srcc, dstc, rows, zbuf, spm, *sems):
        c = lax.axis_index("c")
        s = lax.axis_index("s")
        row0 = s * RPT
        _zero_spmem(zbuf, spm, row0)
        plsc.subcore_barrier()
        grp_base = s * ngrp
        gsem = sems[:GRP]
        ssem = sems[GRP:]

        def chunk(ch, _):
            base = grp_base + ch * CH
            pltpu.sync_copy(src_hbm.at[pl.ds(base, CH)], srcc)
            pltpu.sync_copy(dst_hbm.at[c].at[pl.ds(base, CH)], dstc)

            def group(g, _):
                descs = [pltpu.async_copy(x_hbm.at[srcc.at[g].at[j]],
                                          rows.at[j], gsem[j])
                         for j in range(GRP)]
                scats = []
                for j in range(GRP):
                    descs[j].wait()
                    scats.append(pltpu.async_copy(
                        rows.at[j], spm.at[dstc.at[g].at[j]], ssem[j],
                        add=True))
                for sd in scats:
                    sd.wait()
                return 0
            lax.fori_loop(0, CH, group, 0)
            return 0
        lax.fori_loop(0, NCH, chunk, 0)

        plsc.subcore_barrier()
        pltpu.sync_copy(spm.at[pl.ds(row0, RPT)],
                        out_hbm.at[c].at[pl.ds(row0, RPT)])

    f = pl.kernel(
        body,
        out_type=jax.ShapeDtypeStruct((2, HNPAD, DIM), jnp.float32),
        scratch_types=[
            pltpu.VMEM((CH, GRP, EPB), jnp.int32),
            pltpu.VMEM((CH, GRP, EPB), jnp.int32),
            pltpu.VMEM((GRP, EPB, DIM), jnp.float32),
            pltpu.VMEM((128, DIM), jnp.float32),
            pltpu.VMEM_SHARED((HNPAD, DIM), jnp.float32),
        ] + [pltpu.SemaphoreType.DMA] * (2 * GRP),
        **_MESH,
    )
    return f(src3d, dst4d, x)


def _row_spec():
    return pl.BlockSpec((BN, DIM), lambda i: (i, 0))


def _const_spec(shape):
    return pl.BlockSpec(shape, lambda i: tuple(0 for _ in shape))


def _embed_body(deg_ref, lab_ref, tdeg_ref, tlab_ref, wa_ref, wb_ref,
                lo_ref, hi_ref, t1_ref):
    d = deg_ref[...]
    l = lab_ref[...]
    oh_d = (d == lax.broadcasted_iota(jnp.int32, (BN, 64), 1)).astype(jnp.float32)
    oh_l = (l == lax.broadcasted_iota(jnp.int32, (BN, 16), 1)).astype(jnp.float32)
    lo = jnp.dot(oh_d, tdeg_ref[...], preferred_element_type=jnp.float32)
    hi = jnp.dot(oh_l, tlab_ref[...], preferred_element_type=jnp.float32)
    lo_ref[...] = lo
    hi_ref[...] = hi
    t1_ref[...] = (jnp.dot(lo, wa_ref[...], preferred_element_type=jnp.float32)
                   + jnp.dot(hi, wb_ref[...], preferred_element_type=jnp.float32))


def _embed(nd, nl, embed_deg, embed_lab, wa, wb):
    return pl.pallas_call(
        _embed_body,
        grid=(GRID,),
        in_specs=[pl.BlockSpec((BN, 1), lambda i: (i, 0)),
                  pl.BlockSpec((BN, 1), lambda i: (i, 0)),
                  _const_spec((64, DIM)),
                  _const_spec((16, DIM)),
                  _const_spec((DIM, DIM)),
                  _const_spec((DIM, DIM))],
        out_specs=[_row_spec(), _row_spec(), _row_spec()],
        out_shape=[jax.ShapeDtypeStruct((N, DIM), jnp.float32),
                   jax.ShapeDtypeStruct((N, DIM), jnp.float32),
                   jax.ShapeDtypeStruct((N, DIM), jnp.float32)],
    )(nd, nl, embed_deg, embed_lab, wa, wb)


def _accum_stats(st_ref, y):
    part = jnp.concatenate(
        [jnp.sum(y, 0, keepdims=True), jnp.sum(y * y, 0, keepdims=True)], 0)
    i = pl.program_id(0)

    @pl.when(i == 0)
    def _():
        st_ref[...] = part

    @pl.when(i > 0)
    def _():
        st_ref[...] += part


def _add_stats_body(t_ref, a_ref, b_ref, y_ref, st_ref):
    y = t_ref[...] + a_ref[...] + b_ref[...]
    y_ref[...] = y
    _accum_stats(st_ref, y)


def _add_stats(t, a, b):
    return pl.pallas_call(
        _add_stats_body,
        grid=(GRID,),
        in_specs=[_row_spec(), _row_spec(), _const_spec((1, DIM))],
        out_specs=[_row_spec(), _const_spec((2, DIM))],
        out_shape=[jax.ShapeDtypeStruct((N, DIM), jnp.float32),
                   jax.ShapeDtypeStruct((2, DIM), jnp.float32)],
    )(t, a, b)


def _bn_leaky(y, st_ref, g_ref, be_ref):
    stv = st_ref[...]
    mu = stv[0:1, :] * (1.0 / N)
    var = stv[1:2, :] * (1.0 / N) - mu * mu
    sc = g_ref[...] * lax.rsqrt(var + 1e-5)
    sh = be_ref[...] - mu * sc
    x = y * sc + sh
    return jnp.where(x >= 0, x, 0.01 * x)


def _bnact_mm_body(y_ref, st_ref, g_ref, be_ref, w_ref, x_ref, t_ref):
    x = _bn_leaky(y_ref[...], st_ref, g_ref, be_ref)
    x_ref[...] = x
    t_ref[...] = jnp.dot(x, w_ref[...], preferred_element_type=jnp.float32)


def _bnact_mm(y, st, g, be, w):
    return pl.pallas_call(
        _bnact_mm_body,
        grid=(GRID,),
        in_specs=[_row_spec(), _const_spec((2, DIM)),
                  _const_spec((1, DIM)), _const_spec((1, DIM)),
                  _const_spec((DIM, DIM))],
        out_specs=[_row_spec(), _row_spec()],
        out_shape=[jax.ShapeDtypeStruct((N, DIM), jnp.float32),
                   jax.ShapeDtypeStruct((N, DIM), jnp.float32)],
    )(y, st, g, be, w)


def _final_mm_body(y2_ref, st2_ref, g2_ref, be2_ref, tlo_ref, thi_ref, x1_ref,
                   fa_ref, fb_ref, fc_ref, fd_ref, b_ref, y3_ref, st3_ref):
    x2 = _bn_leaky(y2_ref[...], st2_ref, g2_ref, be2_ref)
    y3 = (jnp.dot(tlo_ref[...], fa_ref[...], preferred_element_type=jnp.float32)
          + jnp.dot(thi_ref[...], fb_ref[...], preferred_element_type=jnp.float32)
          + jnp.dot(x1_ref[...], fc_ref[...], preferred_element_type=jnp.float32)
          + jnp.dot(x2, fd_ref[...], preferred_element_type=jnp.float32)
          + b_ref[...])
    y3_ref[...] = y3
    _accum_stats(st3_ref, y3)


def _final_mm(y2, st2, g2, be2, tlo, thi, x1, fa, fb, fc, fd, b):
    return pl.pallas_call(
        _final_mm_body,
        grid=(GRID,),
        in_specs=[_row_spec(), _const_spec((2, DIM)),
                  _const_spec((1, DIM)), _const_spec((1, DIM)),
                  _row_spec(), _row_spec(), _row_spec(),
                  _const_spec((DIM, DIM)), _const_spec((DIM, DIM)),
                  _const_spec((DIM, DIM)), _const_spec((DIM, DIM)),
                  _const_spec((1, DIM))],
        out_specs=[_row_spec(), _const_spec((2, DIM))],
        out_shape=[jax.ShapeDtypeStruct((N, DIM), jnp.float32),
                   jax.ShapeDtypeStruct((2, DIM), jnp.float32)],
    )(y2, st2, g2, be2, tlo, thi, x1, fa, fb, fc, fd, b)


def _head_body(y3_ref, st3_ref, g_ref, be_ref, w_ref, b_ref, o_ref):
    x3 = _bn_leaky(y3_ref[...], st3_ref, g_ref, be_ref)
    o = jnp.dot(x3, w_ref[...], preferred_element_type=jnp.float32) + b_ref[...]
    o_ref[...] = 1.0 / (1.0 + jnp.exp(-o))


def _head(y3, st3, g, be, w, b):
    return pl.pallas_call(
        _head_body,
        grid=(GRID,),
        in_specs=[_row_spec(), _const_spec((2, DIM)),
                  _const_spec((1, DIM)), _const_spec((1, DIM)),
                  _const_spec((DIM, 1)), _const_spec((1, 1))],
        out_specs=pl.BlockSpec((BN, 1), lambda i: (i, 0)),
        out_shape=jax.ShapeDtypeStruct((N, 1), jnp.float32),
    )(y3, st3, g, be, w, b)


def kernel(node_deg, node_lab, edge_index, embed_deg, embed_lab, W1, b1, g1,
           be1, W2, b2, g2, be2, fcW1, fcb1, fcg, fcbe, fcW2, fcb2):
    nd = node_deg.astype(jnp.int32).reshape(N, 1)
    nl = node_lab.astype(jnp.int32).reshape(N, 1)

    pad = EPAD - E
    src = jnp.concatenate([edge_index[0].astype(jnp.int32),
                           jnp.zeros((pad,), jnp.int32)])
    dst = jnp.concatenate([edge_index[1].astype(jnp.int32),
                           jnp.full((pad,), N, jnp.int32)])
    dst0 = jnp.where(dst < HN, dst, HN)
    dst1 = jnp.where(dst >= HN, dst - HN, HN)
    src3d = src.reshape(NG, GRP, EPB)
    dst4d = jnp.stack([dst0, dst1]).reshape(2, NG, GRP, EPB)

    tlo, thi, t1 = _embed(nd, nl, embed_deg, embed_lab, W1[:DIM], W1[DIM:])

    agg1 = _sc_agg(src3d, dst4d, t1)
    a1 = jnp.concatenate([agg1[0, :HN], agg1[1, :HN]], 0)
    y1, st1 = _add_stats(t1, a1, b1.reshape(1, DIM))
    x1, t2 = _bnact_mm(y1, st1, g1.reshape(1, DIM), be1.reshape(1, DIM), W2)

    agg2 = _sc_agg(src3d, dst4d, t2)
    a2 = jnp.concatenate([agg2[0, :HN], agg2[1, :HN]], 0)
    y2, st2 = _add_stats(t2, a2, b2.reshape(1, DIM))

    y3, st3 = _final_mm(y2, st2, g2.reshape(1, DIM), be2.reshape(1, DIM),
                        tlo, thi, x1,
                        fcW1[0:DIM], fcW1[DIM:2 * DIM],
                        fcW1[2 * DIM:3 * DIM], fcW1[3 * DIM:],
                        fcb1.reshape(1, DIM))
    out = _head(y3, st3, fcg.reshape(1, DIM), fcbe.reshape(1, DIM),
                fcW2, fcb2.reshape(1, 1))
    return out.reshape(N)
